# Initial kernel scaffold; baseline (speedup 1.0000x reference)
import jax
import jax.numpy as jnp
from jax.experimental import pallas as pl


def kernel(x, Wg, bg, W1, b1, W2, b2, gamma, beta):
    raise NotImplementedError("write your pallas kernel here")


# speedup vs baseline: 1.0814x; 1.0814x over previous
import functools

import jax
import jax.numpy as jnp
from jax.experimental import pallas as pl
from jax.experimental.pallas import tpu as pltpu

_EPS = 1e-5


def _moe_block_kernel(
    x_ref, wg_ref, bg_ref, w1_ref, b1_ref, w2_ref, b2_ref, gamma_ref, beta_ref,
    out_ref,
    e0_s, e1_s, w0_s, w1w_s, acc_s,
    *, n_experts,
):
    e = pl.program_id(1)
    x = x_ref[...]

    @pl.when(e == 0)
    def _gate():
        logits = jnp.dot(x, wg_ref[...], preferred_element_type=jnp.float32)
        logits = logits + bg_ref[...]
        eids = jax.lax.broadcasted_iota(jnp.int32, logits.shape, 1)
        v0 = jnp.max(logits, axis=1, keepdims=True)
        e0 = jnp.min(jnp.where(logits == v0, eids, n_experts), axis=1, keepdims=True)
        masked = jnp.where(eids == e0, -jnp.inf, logits)
        v1 = jnp.max(masked, axis=1, keepdims=True)
        e1 = jnp.min(jnp.where(masked == v1, eids, n_experts), axis=1, keepdims=True)
        w0 = 1.0 / (1.0 + jnp.exp(v1 - v0))
        e0_s[...] = e0
        e1_s[...] = e1
        w0_s[...] = w0
        w1w_s[...] = 1.0 - w0

    coef = (w0_s[...] * (e0_s[...] == e).astype(jnp.float32)
            + w1w_s[...] * (e1_s[...] == e).astype(jnp.float32))

    xb = x.astype(jnp.bfloat16)
    h = jnp.dot(xb, w1_ref[0], preferred_element_type=jnp.float32) + b1_ref[0]
    h = jnp.maximum(h, 0.0).astype(jnp.bfloat16)
    y = jnp.dot(h, w2_ref[0], preferred_element_type=jnp.float32) + b2_ref[0]
    contrib = coef * y

    @pl.when(e == 0)
    def _init():
        acc_s[...] = contrib

    @pl.when(e > 0)
    def _accum():
        acc_s[...] = acc_s[...] + contrib

    @pl.when(e == n_experts - 1)
    def _finish():
        z = x + acc_s[...]
        mean = jnp.mean(z, axis=1, keepdims=True)
        zc = z - mean
        var = jnp.mean(zc * zc, axis=1, keepdims=True)
        zn = zc * jax.lax.rsqrt(var + _EPS)
        out_ref[...] = zn * gamma_ref[...] + beta_ref[...]


def kernel(x, Wg, bg, W1, b1, W2, b2, gamma, beta):
    B, S, H = x.shape
    E = Wg.shape[1]
    F = W1.shape[2]
    N = B * S
    RB = 256
    NB = N // RB

    xf = x.reshape(N, H)
    w1b = W1.astype(jnp.bfloat16)
    w2b = W2.astype(jnp.bfloat16)

    out = pl.pallas_call(
        functools.partial(_moe_block_kernel, n_experts=E),
        grid=(NB, E),
        in_specs=[
            pl.BlockSpec((RB, H), lambda i, e: (i, 0)),
            pl.BlockSpec((H, E), lambda i, e: (0, 0)),
            pl.BlockSpec((1, E), lambda i, e: (0, 0)),
            pl.BlockSpec((1, H, F), lambda i, e: (e, 0, 0)),
            pl.BlockSpec((1, 1, F), lambda i, e: (e, 0, 0)),
            pl.BlockSpec((1, F, H), lambda i, e: (e, 0, 0)),
            pl.BlockSpec((1, 1, H), lambda i, e: (e, 0, 0)),
            pl.BlockSpec((1, H), lambda i, e: (0, 0)),
            pl.BlockSpec((1, H), lambda i, e: (0, 0)),
        ],
        out_specs=pl.BlockSpec((RB, H), lambda i, e: (i, 0)),
        out_shape=jax.ShapeDtypeStruct((N, H), jnp.float32),
        scratch_shapes=[
            pltpu.VMEM((RB, 1), jnp.int32),
            pltpu.VMEM((RB, 1), jnp.int32),
            pltpu.VMEM((RB, 1), jnp.float32),
            pltpu.VMEM((RB, 1), jnp.float32),
            pltpu.VMEM((RB, H), jnp.float32),
        ],
        compiler_params=pltpu.CompilerParams(
            dimension_semantics=("arbitrary", "arbitrary"),
        ),
    )(xf, Wg, bg.reshape(1, E), w1b, b1.reshape(E, 1, F), w2b,
      b2.reshape(E, 1, H), gamma.reshape(1, H), beta.reshape(1, H))

    return out.reshape(B, S, H)


# speedup vs baseline: 1.8356x; 1.6975x over previous
import functools

import jax
import jax.numpy as jnp
from jax import lax
from jax.experimental import pallas as pl
from jax.experimental.pallas import tpu as pltpu
from jax.experimental.pallas import tpu_sc as plsc

_EPS = 1e-5


def _gate_kernel(x_ref, wg_ref, bg_ref, e0_ref, e1_ref, w0_ref, w1_ref,
                 *, n_experts):
    logits = jnp.dot(x_ref[...], wg_ref[...], preferred_element_type=jnp.float32)
    logits = logits + bg_ref[...]
    eids = jax.lax.broadcasted_iota(jnp.int32, logits.shape, 1)
    v0 = jnp.max(logits, axis=1, keepdims=True)
    e0 = jnp.min(jnp.where(logits == v0, eids, n_experts), axis=1, keepdims=True)
    masked = jnp.where(eids == e0, -jnp.inf, logits)
    v1 = jnp.max(masked, axis=1, keepdims=True)
    e1 = jnp.min(jnp.where(masked == v1, eids, n_experts), axis=1, keepdims=True)
    w0 = 1.0 / (1.0 + jnp.exp(v1 - v0))
    e0_ref[...] = e0
    e1_ref[...] = e1
    w0_ref[...] = w0
    w1_ref[...] = 1.0 - w0


def _gate(xf, Wg, bg, n, h, e):
    rb = 512
    return pl.pallas_call(
        functools.partial(_gate_kernel, n_experts=e),
        grid=(n // rb,),
        in_specs=[
            pl.BlockSpec((rb, h), lambda i: (i, 0)),
            pl.BlockSpec((h, e), lambda i: (0, 0)),
            pl.BlockSpec((1, e), lambda i: (0, 0)),
        ],
        out_specs=[
            pl.BlockSpec((rb, 1), lambda i: (i, 0)),
            pl.BlockSpec((rb, 1), lambda i: (i, 0)),
            pl.BlockSpec((rb, 1), lambda i: (i, 0)),
            pl.BlockSpec((rb, 1), lambda i: (i, 0)),
        ],
        out_shape=[
            jax.ShapeDtypeStruct((n, 1), jnp.int32),
            jax.ShapeDtypeStruct((n, 1), jnp.int32),
            jax.ShapeDtypeStruct((n, 1), jnp.float32),
            jax.ShapeDtypeStruct((n, 1), jnp.float32),
        ],
    )(xf, Wg, bg.reshape(1, e))


def _sc_gather_rows(table, idx):
    t, d = table.shape
    p = idx.shape[0]
    nw = 32
    rows_w = p // nw
    ch = 32
    nch = rows_w // ch
    mesh = plsc.VectorSubcoreMesh(core_axis_name="c", subcore_axis_name="s")

    @functools.partial(
        pl.kernel,
        mesh=mesh,
        out_type=jax.ShapeDtypeStruct((p, d), jnp.float32),
        scratch_types=[
            pltpu.VMEM((rows_w,), jnp.int32),
            pltpu.VMEM((ch, d), jnp.float32),
            pltpu.VMEM((ch, d), jnp.float32),
            pltpu.SemaphoreType.DMA,
            pltpu.SemaphoreType.DMA,
        ],
    )
    def k(table_hbm, idx_hbm, out_hbm, idx_v, buf0, buf1, sem0, sem1):
        wid = lax.axis_index("s") * 2 + lax.axis_index("c")
        base = wid * rows_w
        pltpu.sync_copy(idx_hbm.at[pl.ds(base, rows_w)], idx_v)
        bufs = (buf0, buf1)
        sems = (sem0, sem1)
        pltpu.async_copy(table_hbm.at[idx_v.at[pl.ds(0, ch)]], bufs[0], sems[0])
        for c in range(nch):
            cur, sem = bufs[c % 2], sems[c % 2]
            if c + 1 < nch:
                pltpu.async_copy(
                    table_hbm.at[idx_v.at[pl.ds((c + 1) * ch, ch)]],
                    bufs[(c + 1) % 2], sems[(c + 1) % 2])
            pltpu.make_async_copy(
                table_hbm.at[idx_v.at[pl.ds(c * ch, ch)]], cur, sem).wait()
            pltpu.sync_copy(cur, out_hbm.at[pl.ds(base + c * ch, ch)])

    return k(table, idx)


def _gmm_kernel(sb_ref, se_ref, gs_ref, ge_ref,
                xs_ref, w1_ref, b1_ref, w2_ref, b2_ref, out_ref, *, r):
    g = pl.program_id(0)
    b = sb_ref[g]
    rows = b * r + jax.lax.broadcasted_iota(jnp.int32, (r, 1), 0)
    mask = (rows >= gs_ref[g]) & (rows < ge_ref[g])

    a = xs_ref[...].astype(jnp.bfloat16)
    h = jnp.dot(a, w1_ref[0], preferred_element_type=jnp.float32) + b1_ref[0]
    h = jnp.maximum(h, 0.0).astype(jnp.bfloat16)
    y = jnp.dot(h, w2_ref[0], preferred_element_type=jnp.float32) + b2_ref[0]

    first = sb_ref[jnp.maximum(g - 1, 0)] != b
    first = jnp.logical_or(g == 0, first)

    @pl.when(first)
    def _():
        out_ref[...] = jnp.where(mask, y, 0.0)

    @pl.when(jnp.logical_not(first))
    def _():
        out_ref[...] = jnp.where(mask, y, out_ref[...])


def _grouped_ffn(xs, w1b, b1, w2b, b2, sb, se, gs, ge, n_steps, r, h, f, e):
    p = xs.shape[0]
    grid_spec = pltpu.PrefetchScalarGridSpec(
        num_scalar_prefetch=4,
        grid=(n_steps,),
        in_specs=[
            pl.BlockSpec((r, h), lambda g, sb, se, gs, ge: (sb[g], 0)),
            pl.BlockSpec((1, h, f), lambda g, sb, se, gs, ge: (se[g], 0, 0)),
            pl.BlockSpec((1, 1, f), lambda g, sb, se, gs, ge: (se[g], 0, 0)),
            pl.BlockSpec((1, f, h), lambda g, sb, se, gs, ge: (se[g], 0, 0)),
            pl.BlockSpec((1, 1, h), lambda g, sb, se, gs, ge: (se[g], 0, 0)),
        ],
        out_specs=pl.BlockSpec((r, h), lambda g, sb, se, gs, ge: (sb[g], 0)),
    )
    return pl.pallas_call(
        functools.partial(_gmm_kernel, r=r),
        grid_spec=grid_spec,
        out_shape=jax.ShapeDtypeStruct((p, h), jnp.float32),
        compiler_params=pltpu.CompilerParams(
            dimension_semantics=("arbitrary",),
        ),
    )(sb, se, gs, ge, xs, w1b, b1.reshape(e, 1, f), w2b, b2.reshape(e, 1, h))


def _combine_ln_kernel(x_ref, y0_ref, y1_ref, w0_ref, w1_ref,
                       gamma_ref, beta_ref, out_ref):
    z = (x_ref[...] + w0_ref[...] * y0_ref[...] + w1_ref[...] * y1_ref[...])
    mean = jnp.mean(z, axis=1, keepdims=True)
    zc = z - mean
    var = jnp.mean(zc * zc, axis=1, keepdims=True)
    out_ref[...] = zc * jax.lax.rsqrt(var + _EPS) * gamma_ref[...] + beta_ref[...]


def _combine_ln(xf, yg, w0, w1, gamma, beta, n, h):
    rb = 512
    nb = n // rb
    return pl.pallas_call(
        _combine_ln_kernel,
        grid=(nb,),
        in_specs=[
            pl.BlockSpec((rb, h), lambda i: (i, 0)),
            pl.BlockSpec((rb, h), lambda i: (i, 0)),
            pl.BlockSpec((rb, h), lambda i: (i + nb, 0)),
            pl.BlockSpec((rb, 1), lambda i: (i, 0)),
            pl.BlockSpec((rb, 1), lambda i: (i, 0)),
            pl.BlockSpec((1, h), lambda i: (0, 0)),
            pl.BlockSpec((1, h), lambda i: (0, 0)),
        ],
        out_specs=pl.BlockSpec((rb, h), lambda i: (i, 0)),
        out_shape=jax.ShapeDtypeStruct((n, h), jnp.float32),
    )(xf, yg, yg, w0, w1, gamma.reshape(1, h), beta.reshape(1, h))


def kernel(x, Wg, bg, W1, b1, W2, b2, gamma, beta):
    B, S, H = x.shape
    E = Wg.shape[1]
    F = W1.shape[2]
    N = B * S
    P = 2 * N
    R = 256
    M = P // R
    G = M + E - 1

    xf = x.reshape(N, H)
    w1b = W1.astype(jnp.bfloat16)
    w2b = W2.astype(jnp.bfloat16)

    e0c, e1c, w0c, w1c = _gate(xf, Wg, bg, N, H, E)
    e0 = e0c[:, 0]
    e1 = e1c[:, 0]

    pe = jnp.concatenate([e0, e1])
    onehot = (pe[:, None] == jnp.arange(E)[None, :])
    counts = jnp.sum(onehot, axis=0, dtype=jnp.int32)
    ends = jnp.cumsum(counts)
    starts = ends - counts
    rank = jnp.cumsum(onehot.astype(jnp.int32), axis=0) - onehot.astype(jnp.int32)
    pos = starts[pe] + jnp.sum(jnp.where(onehot, rank, 0), axis=1)
    arange_p = jnp.arange(P, dtype=jnp.int32)
    sort_idx = jnp.zeros((P,), jnp.int32).at[pos].set(arange_p)
    st = (sort_idx % N).astype(jnp.int32)
    inv = pos.astype(jnp.int32)

    bidx = jnp.arange(M, dtype=jnp.int32)
    present = ((starts[None, :] < (bidx[:, None] + 1) * R)
               & (ends[None, :] > bidx[:, None] * R))
    flat = jnp.nonzero(present.ravel(), size=G, fill_value=M * E - 1)[0]
    flat = flat.astype(jnp.int32)
    sb = flat // E
    se = flat % E
    gs = starts[se].astype(jnp.int32)
    ge = ends[se].astype(jnp.int32)

    xs = _sc_gather_rows(xf, st)
    ys = _grouped_ffn(xs, w1b, b1, w2b, b2, sb, se, gs, ge, G, R, H, F, E)
    yg = _sc_gather_rows(ys, inv)
    out = _combine_ln(xf, yg, w0c, w1c, gamma, beta, N, H)
    return out.reshape(B, S, H)
